# baseline (speedup 1.0000x reference)
import jax
import jax.numpy as jnp
from jax.experimental import pallas as pl
from jax.experimental.pallas import tpu as pltpu

_B, _C, _H, _W = 32, 96, 56, 56
_E = 8
_KR = _C * _H * _W
_NKC = 8
_KC = _KR // _NKC
_HW = _H * _W
_PAD = 57


def _router_kernel(x_ref, wr_ref, loss_ref, sel_ref, x16_ref, acc_ref):
    k = pl.program_id(0)
    xb = x_ref[...]
    x16_ref[...] = xb.astype(jnp.bfloat16)

    @pl.when(k == 0)
    def _init():
        acc_ref[...] = jnp.zeros_like(acc_ref)

    acc_ref[...] += jax.lax.dot_general(
        xb, wr_ref[...], (((1,), (1,)), ((), ())),
        preferred_element_type=jnp.float32)

    @pl.when(k == pl.num_programs(0) - 1)
    def _fini():
        _router_tail(acc_ref, loss_ref, sel_ref)


def _router_tail(acc_ref, loss_ref, sel_ref):
    logits = acc_ref[...]
    m = jnp.max(logits, axis=1, keepdims=True)
    ex = jnp.exp(logits - m)
    p = ex / jnp.sum(ex, axis=1, keepdims=True)
    avg = jnp.mean(p, axis=0, keepdims=True)
    d = avg - jnp.float32(1.0 / _E)
    loss_ref[...] = jnp.mean(d * d, axis=1, keepdims=True)

    row = logits[0:1, :]
    col = jax.lax.broadcasted_iota(jnp.int32, (1, _E), 1)
    m0 = jnp.max(row, axis=1, keepdims=True)
    i0 = jnp.min(jnp.where(row == m0, col, _E), axis=1, keepdims=True)
    row1 = jnp.where(col == i0, -jnp.inf, row)
    m1 = jnp.max(row1, axis=1, keepdims=True)
    i1 = jnp.min(jnp.where(row1 == m1, col, _E), axis=1, keepdims=True)
    sel_ref[...] = jnp.concatenate([i0, i1], axis=1)


def _conv_kernel(sel_ref, x_ref, wa_ref, wb_ref, ba_ref, bb_ref, out_ref,
                 wt_ref):
    del sel_ref
    b = pl.program_id(0)

    @pl.when(b == 0)
    def _unpack_taps():
        for i, w_ref in enumerate((wa_ref, wb_ref)):
            w3 = w_ref[0]
            for t in range(9):
                wt_ref[i, t] = w3[:, :, t].astype(jnp.bfloat16)

    x2 = x_ref[0]
    col = jax.lax.broadcasted_iota(jnp.int32, (1, _HW), 1) % _W
    x2l = jnp.where(col == _W - 1, jnp.bfloat16(0), x2)
    x2r = jnp.where(col == 0, jnp.bfloat16(0), x2)
    xe = jnp.pad(x2, ((0, 0), (_PAD, _PAD)))
    xel = jnp.pad(x2l, ((0, 0), (_PAD, _PAD)))
    xer = jnp.pad(x2r, ((0, 0), (_PAD, _PAD)))
    srcs = (xel, xe, xer)

    acc_a = jnp.zeros((_C, _HW), jnp.float32)
    acc_b = jnp.zeros((_C, _HW), jnp.float32)
    for dy in range(3):
        for dx in range(3):
            s = (dy - 1) * _W + (dx - 1)
            xs = srcs[dx][:, _PAD + s:_PAD + s + _HW]
            acc_a = acc_a + jnp.dot(wt_ref[0, dy * 3 + dx], xs,
                                    preferred_element_type=jnp.float32)
            acc_b = acc_b + jnp.dot(wt_ref[1, dy * 3 + dx], xs,
                                    preferred_element_type=jnp.float32)
    out_ref[0] = jnp.concatenate(
        [acc_a + ba_ref[0], acc_b + bb_ref[0]], axis=0)


def kernel(x, W_router, W_conv, b_conv):
    xf = x.reshape(_B, _KR)

    loss2, sel2, x16 = pl.pallas_call(
        _router_kernel,
        grid=(_NKC,),
        in_specs=[
            pl.BlockSpec((_B, _KC), lambda k: (0, k)),
            pl.BlockSpec((_E, _KC), lambda k: (0, k)),
        ],
        out_specs=[
            pl.BlockSpec((1, 1), lambda k: (0, 0)),
            pl.BlockSpec((1, 2), lambda k: (0, 0)),
            pl.BlockSpec((_B, _KC), lambda k: (0, k)),
        ],
        out_shape=[
            jax.ShapeDtypeStruct((1, 1), jnp.float32),
            jax.ShapeDtypeStruct((1, 2), jnp.int32),
            jax.ShapeDtypeStruct((_B, _KR), jnp.bfloat16),
        ],
        scratch_shapes=[pltpu.VMEM((_B, _E), jnp.float32)],
    )(xf, W_router)
    sel = sel2.reshape(2)
    router_loss = loss2.reshape(())

    x16f = x16.reshape(_B, _C, _HW)
    w_r = W_conv.reshape(_E, _C, _C, 9)
    b_r = b_conv.reshape(_E, _C, 1)

    grid_spec = pltpu.PrefetchScalarGridSpec(
        num_scalar_prefetch=1,
        grid=(_B,),
        in_specs=[
            pl.BlockSpec((1, _C, _HW), lambda b, s: (b, 0, 0)),
            pl.BlockSpec((1, _C, _C, 9), lambda b, s: (s[0], 0, 0, 0)),
            pl.BlockSpec((1, _C, _C, 9), lambda b, s: (s[1], 0, 0, 0)),
            pl.BlockSpec((1, _C, 1), lambda b, s: (s[0], 0, 0)),
            pl.BlockSpec((1, _C, 1), lambda b, s: (s[1], 0, 0)),
        ],
        out_specs=pl.BlockSpec((1, 2 * _C, _HW), lambda b, s: (b, 0, 0)),
        scratch_shapes=[pltpu.VMEM((2, 9, _C, _C), jnp.bfloat16)],
    )
    out_raw = pl.pallas_call(
        _conv_kernel,
        grid_spec=grid_spec,
        out_shape=jax.ShapeDtypeStruct((_B, 2 * _C, _HW), jnp.float32),
    )(sel, x16f, w_r, w_r, b_r, b_r)

    expert_outputs = out_raw.reshape(_B, 2 * _C, _H, _W)
    return expert_outputs, router_loss


# speedup vs baseline: 1.4875x; 1.4875x over previous
import jax
import jax.numpy as jnp
from jax.experimental import pallas as pl
from jax.experimental.pallas import tpu as pltpu

_B, _C, _H, _W = 32, 96, 56, 56
_E = 8
_HW = _H * _W
_WC = _C * 9
_EC = _E * _C
_PAD = 57


def _fused_kernel(x_ref, wr_ref, wc_ref, bc_ref,
                  loss_ref, sel_ref, out_ref, wt_ref, bsel_ref, logits_ref):
    b = pl.program_id(0)

    @pl.when(b < _E)
    def _router_partial():
        wre = wr_ref[0]
        colr = jax.lax.broadcasted_iota(jnp.int32, (1, _E), 1)

        def _one(i, _):
            x2 = x_ref[i]
            s = jnp.sum(jnp.sum(x2 * wre, axis=1, keepdims=True),
                        axis=0, keepdims=True)
            row_old = logits_ref[pl.ds(i, 1), :]
            logits_ref[pl.ds(i, 1), :] = jnp.where(colr == b, s, row_old)
            return 0

        jax.lax.fori_loop(0, _B, _one, 0)

    @pl.when(b == _E - 1)
    def _router_tail_and_gather():
        logits = logits_ref[...]
        m = jnp.max(logits, axis=1, keepdims=True)
        ex = jnp.exp(logits - m)
        p = ex / jnp.sum(ex, axis=1, keepdims=True)
        avg = jnp.mean(p, axis=0, keepdims=True)
        d = avg - jnp.float32(1.0 / _E)
        loss_ref[...] = jnp.mean(d * d, axis=1, keepdims=True)

        row = logits[0:1, :]
        col = jax.lax.broadcasted_iota(jnp.int32, (1, _E), 1)
        m0 = jnp.max(row, axis=1, keepdims=True)
        i0 = jnp.min(jnp.where(row == m0, col, _E), axis=1, keepdims=True)
        row1 = jnp.where(col == i0, -jnp.inf, row)
        m1 = jnp.max(row1, axis=1, keepdims=True)
        i1 = jnp.min(jnp.where(row1 == m1, col, _E), axis=1, keepdims=True)
        sel_ref[...] = jnp.concatenate([i0, i1], axis=1)

        rr = jax.lax.broadcasted_iota(jnp.int32, (_C, _EC), 0)
        rk = jax.lax.broadcasted_iota(jnp.int32, (_C, _EC), 1)
        kr = jax.lax.broadcasted_iota(jnp.int32, (_WC, _C), 0)
        kc = jax.lax.broadcasted_iota(jnp.int32, (_WC, _C), 1)
        for slot, idx in ((0, i0), (1, i1)):
            sel_rows = (rk == idx * _C + rr).astype(jnp.float32)
            w2 = jnp.dot(sel_rows, wc_ref[...],
                         preferred_element_type=jnp.float32)
            oh = (col == idx).astype(jnp.float32)
            bvec = jnp.dot(oh, bc_ref[...],
                           preferred_element_type=jnp.float32)
            bsel_ref[slot] = jnp.transpose(bvec)
            w2b = w2.astype(jnp.bfloat16)
            for t in range(9):
                s_t = (kr == kc * 9 + t).astype(jnp.bfloat16)
                wt_ref[slot, t] = jnp.dot(
                    w2b, s_t, preferred_element_type=jnp.float32
                ).astype(jnp.bfloat16)

    @pl.when(b >= _E)
    def _conv():
        i = b - _E
        x2 = x_ref[i].astype(jnp.bfloat16)
        ccol = jax.lax.broadcasted_iota(jnp.int32, (1, _HW), 1) % _W
        zero = jnp.bfloat16(0)
        x2l = jnp.where(ccol == _W - 1, zero, x2)
        x2r = jnp.where(ccol == 0, zero, x2)
        xe = jnp.pad(x2, ((0, 0), (_PAD, _PAD)))
        xel = jnp.pad(x2l, ((0, 0), (_PAD, _PAD)))
        xer = jnp.pad(x2r, ((0, 0), (_PAD, _PAD)))
        srcs = (xel, xe, xer)

        acc_a = jnp.zeros((_C, _HW), jnp.float32)
        acc_b = jnp.zeros((_C, _HW), jnp.float32)
        for dy in range(3):
            for dx in range(3):
                s = (dy - 1) * _W + (dx - 1)
                xs = srcs[dx][:, _PAD + s:_PAD + s + _HW]
                acc_a = acc_a + jnp.dot(wt_ref[0, dy * 3 + dx], xs,
                                        preferred_element_type=jnp.float32)
                acc_b = acc_b + jnp.dot(wt_ref[1, dy * 3 + dx], xs,
                                        preferred_element_type=jnp.float32)
        out_ref[0] = jnp.concatenate(
            [acc_a + bsel_ref[0], acc_b + bsel_ref[1]],
            axis=0)


def kernel(x, W_router, W_conv, b_conv):
    x3 = x.reshape(_B, _C, _HW)
    wr3 = W_router.reshape(_E, _C, _HW)
    wc2 = W_conv.reshape(_EC, _WC)

    loss2, sel2, out_raw = pl.pallas_call(
        _fused_kernel,
        grid=(_B + _E,),
        in_specs=[
            pl.BlockSpec((_B, _C, _HW), lambda b: (0, 0, 0)),
            pl.BlockSpec((1, _C, _HW),
                         lambda b: (jnp.minimum(b, _E - 1), 0, 0)),
            pl.BlockSpec((_EC, _WC), lambda b: (0, 0)),
            pl.BlockSpec((_E, _C), lambda b: (0, 0)),
        ],
        out_specs=[
            pl.BlockSpec((1, 1), lambda b: (0, 0)),
            pl.BlockSpec((1, 2), lambda b: (0, 0)),
            pl.BlockSpec((1, 2 * _C, _HW),
                         lambda b: (jnp.maximum(b - _E, 0), 0, 0)),
        ],
        out_shape=[
            jax.ShapeDtypeStruct((1, 1), jnp.float32),
            jax.ShapeDtypeStruct((1, 2), jnp.int32),
            jax.ShapeDtypeStruct((_B, 2 * _C, _HW), jnp.float32),
        ],
        scratch_shapes=[
            pltpu.VMEM((2, 9, _C, _C), jnp.bfloat16),
            pltpu.VMEM((2, _C, 1), jnp.float32),
            pltpu.VMEM((_B, _E), jnp.float32),
        ],
    )(x3, wr3, wc2, b_conv)

    router_loss = loss2.reshape(())
    expert_outputs = out_raw.reshape(_B, 2 * _C, _H, _W)
    return expert_outputs, router_loss


# speedup vs baseline: 1.5579x; 1.0473x over previous
import jax
import jax.numpy as jnp
from jax.experimental import pallas as pl
from jax.experimental.pallas import tpu as pltpu

_B, _C, _H, _W = 32, 96, 56, 56
_E = 8
_HW = _H * _W
_WC = _C * 9
_EC = _E * _C
_PAD = 57
_CC = 16
_NC = _C // _CC


def _fused_kernel(x_ref, wr_ref, wc_ref, bc_ref,
                  loss_ref, sel_ref, out_ref,
                  xs_ref, wt_ref, bsel_ref, logits_ref):
    b = pl.program_id(0)

    @pl.when(b < _NC)
    def _stream_and_route():
        xblk = x_ref[...]
        xs_ref[:, pl.ds(b * _CC, _CC), :] = xblk.astype(jnp.bfloat16)

        @pl.when(b == 0)
        def _init():
            logits_ref[...] = jnp.zeros_like(logits_ref)

        def _one(i, _):
            x2 = x_ref[i]
            parts = []
            for e in range(_E):
                pr = x2 * wr_ref[e]
                s = jnp.sum(jnp.sum(pr, axis=1, keepdims=True),
                            axis=0, keepdims=True)
                parts.append(s)
            row = jnp.concatenate(parts, axis=1)
            logits_ref[pl.ds(i, 1), :] = logits_ref[pl.ds(i, 1), :] + row
            return 0

        jax.lax.fori_loop(0, _B, _one, 0)

    @pl.when(b == _NC - 1)
    def _router_tail_and_gather():
        logits = logits_ref[...]
        m = jnp.max(logits, axis=1, keepdims=True)
        ex = jnp.exp(logits - m)
        p = ex / jnp.sum(ex, axis=1, keepdims=True)
        avg = jnp.mean(p, axis=0, keepdims=True)
        d = avg - jnp.float32(1.0 / _E)
        loss_ref[...] = jnp.mean(d * d, axis=1, keepdims=True)

        row = logits[0:1, :]
        col = jax.lax.broadcasted_iota(jnp.int32, (1, _E), 1)
        m0 = jnp.max(row, axis=1, keepdims=True)
        i0 = jnp.min(jnp.where(row == m0, col, _E), axis=1, keepdims=True)
        row1 = jnp.where(col == i0, -jnp.inf, row)
        m1 = jnp.max(row1, axis=1, keepdims=True)
        i1 = jnp.min(jnp.where(row1 == m1, col, _E), axis=1, keepdims=True)
        sel_ref[...] = jnp.concatenate([i0, i1], axis=1)

        rr = jax.lax.broadcasted_iota(jnp.int32, (_C, _EC), 0)
        rk = jax.lax.broadcasted_iota(jnp.int32, (_C, _EC), 1)
        kr = jax.lax.broadcasted_iota(jnp.int32, (_WC, _C), 0)
        kc = jax.lax.broadcasted_iota(jnp.int32, (_WC, _C), 1)
        for slot, idx in ((0, i0), (1, i1)):
            sel_rows = (rk == idx * _C + rr).astype(jnp.float32)
            w2 = jnp.dot(sel_rows, wc_ref[...],
                         preferred_element_type=jnp.float32)
            oh = (col == idx).astype(jnp.float32)
            bvec = jnp.dot(oh, bc_ref[...],
                           preferred_element_type=jnp.float32)
            bsel_ref[slot] = jnp.transpose(bvec)
            w2b = w2.astype(jnp.bfloat16)
            for t in range(9):
                s_t = (kr == kc * 9 + t).astype(jnp.bfloat16)
                wt_ref[slot, t] = jnp.dot(
                    w2b, s_t, preferred_element_type=jnp.float32
                ).astype(jnp.bfloat16)

    @pl.when(b >= _NC)
    def _conv():
        i = b - _NC
        x2 = xs_ref[i]
        ccol = jax.lax.broadcasted_iota(jnp.int32, (1, _HW), 1) % _W
        zero = jnp.bfloat16(0)
        x2l = jnp.where(ccol == _W - 1, zero, x2)
        x2r = jnp.where(ccol == 0, zero, x2)
        xe = jnp.pad(x2, ((0, 0), (_PAD, _PAD)))
        xel = jnp.pad(x2l, ((0, 0), (_PAD, _PAD)))
        xer = jnp.pad(x2r, ((0, 0), (_PAD, _PAD)))
        srcs = (xel, xe, xer)

        acc_a = jnp.zeros((_C, _HW), jnp.float32)
        acc_b = jnp.zeros((_C, _HW), jnp.float32)
        for dy in range(3):
            for dx in range(3):
                s = (dy - 1) * _W + (dx - 1)
                xsl = srcs[dx][:, _PAD + s:_PAD + s + _HW]
                acc_a = acc_a + jnp.dot(wt_ref[0, dy * 3 + dx], xsl,
                                        preferred_element_type=jnp.float32)
                acc_b = acc_b + jnp.dot(wt_ref[1, dy * 3 + dx], xsl,
                                        preferred_element_type=jnp.float32)
        out_ref[0] = jnp.concatenate(
            [acc_a + bsel_ref[0], acc_b + bsel_ref[1]],
            axis=0)


def kernel(x, W_router, W_conv, b_conv):
    x3 = x.reshape(_B, _C, _HW)
    wr3 = W_router.reshape(_E, _C, _HW)
    wc2 = W_conv.reshape(_EC, _WC)

    loss2, sel2, out_raw = pl.pallas_call(
        _fused_kernel,
        grid=(_B + _NC,),
        in_specs=[
            pl.BlockSpec((_B, _CC, _HW),
                         lambda b: (0, jnp.minimum(b, _NC - 1), 0)),
            pl.BlockSpec((_E, _CC, _HW),
                         lambda b: (0, jnp.minimum(b, _NC - 1), 0)),
            pl.BlockSpec((_EC, _WC), lambda b: (0, 0)),
            pl.BlockSpec((_E, _C), lambda b: (0, 0)),
        ],
        out_specs=[
            pl.BlockSpec((1, 1), lambda b: (0, 0)),
            pl.BlockSpec((1, 2), lambda b: (0, 0)),
            pl.BlockSpec((1, 2 * _C, _HW),
                         lambda b: (jnp.maximum(b - _NC, 0), 0, 0)),
        ],
        out_shape=[
            jax.ShapeDtypeStruct((1, 1), jnp.float32),
            jax.ShapeDtypeStruct((1, 2), jnp.int32),
            jax.ShapeDtypeStruct((_B, 2 * _C, _HW), jnp.float32),
        ],
        scratch_shapes=[
            pltpu.VMEM((_B, _C, _HW), jnp.bfloat16),
            pltpu.VMEM((2, 9, _C, _C), jnp.bfloat16),
            pltpu.VMEM((2, _C, 1), jnp.float32),
            pltpu.VMEM((_B, _E), jnp.float32),
        ],
    )(x3, wr3, wc2, b_conv)

    router_loss = loss2.reshape(())
    expert_outputs = out_raw.reshape(_B, 2 * _C, _H, _W)
    return expert_outputs, router_loss


# speedup vs baseline: 1.5820x; 1.0155x over previous
import jax
import jax.numpy as jnp
from jax.experimental import pallas as pl
from jax.experimental.pallas import tpu as pltpu

_B, _C, _H, _W = 32, 96, 56, 56
_E = 8
_HW = _H * _W
_WC = _C * 9
_EC = _E * _C
_PAD = 57
_CC = 16
_NC = _C // _CC


def _fused_kernel(x_ref, wr_ref, wc_ref, bc_ref,
                  loss_ref, sel_ref, out_ref,
                  xs_ref, wt_ref, bsel_ref, logits_ref):
    b = pl.program_id(0)

    @pl.when(b < _NC)
    def _stream_and_route():
        xblk = x_ref[...]
        xs_ref[:, pl.ds(b * _CC, _CC), :] = xblk.astype(jnp.bfloat16)

        @pl.when(b == 0)
        def _init():
            logits_ref[...] = jnp.zeros_like(logits_ref)

        def _one(i, _):
            x2 = x_ref[i]
            parts = []
            for e in range(_E):
                pr = x2 * wr_ref[e]
                s = jnp.sum(jnp.sum(pr, axis=1, keepdims=True),
                            axis=0, keepdims=True)
                parts.append(s)
            row = jnp.concatenate(parts, axis=1)
            logits_ref[pl.ds(i, 1), :] = logits_ref[pl.ds(i, 1), :] + row
            return 0

        jax.lax.fori_loop(0, _B, _one, 0)

    @pl.when(b == _NC - 1)
    def _router_tail_and_gather():
        logits = logits_ref[...]
        m = jnp.max(logits, axis=1, keepdims=True)
        ex = jnp.exp(logits - m)
        p = ex / jnp.sum(ex, axis=1, keepdims=True)
        avg = jnp.mean(p, axis=0, keepdims=True)
        d = avg - jnp.float32(1.0 / _E)
        loss_ref[...] = jnp.mean(d * d, axis=1, keepdims=True)

        row = logits[0:1, :]
        col = jax.lax.broadcasted_iota(jnp.int32, (1, _E), 1)
        m0 = jnp.max(row, axis=1, keepdims=True)
        i0 = jnp.min(jnp.where(row == m0, col, _E), axis=1, keepdims=True)
        row1 = jnp.where(col == i0, -jnp.inf, row)
        m1 = jnp.max(row1, axis=1, keepdims=True)
        i1 = jnp.min(jnp.where(row1 == m1, col, _E), axis=1, keepdims=True)
        sel_ref[...] = jnp.concatenate([i0, i1], axis=1)

        rr = jax.lax.broadcasted_iota(jnp.int32, (_C, _EC), 0)
        rk = jax.lax.broadcasted_iota(jnp.int32, (_C, _EC), 1)
        kr = jax.lax.broadcasted_iota(jnp.int32, (_WC, _C), 0)
        kc = jax.lax.broadcasted_iota(jnp.int32, (_WC, _C), 1)
        for slot, idx in ((0, i0), (1, i1)):
            sel_rows = (rk == idx * _C + rr).astype(jnp.float32)
            w2 = jnp.dot(sel_rows, wc_ref[...],
                         preferred_element_type=jnp.float32)
            oh = (col == idx).astype(jnp.float32)
            bvec = jnp.dot(oh, bc_ref[...],
                           preferred_element_type=jnp.float32)
            bsel_ref[slot] = jnp.transpose(bvec)
            w2b = w2.astype(jnp.bfloat16)
            for t in range(9):
                s_t = (kr == kc * 9 + t).astype(jnp.bfloat16)
                wt_ref[slot, t] = jnp.dot(
                    w2b, s_t, preferred_element_type=jnp.float32
                ).astype(jnp.bfloat16)

    @pl.when(b >= _NC)
    def _conv():
      for s2 in range(2):
          i = (b - _NC) * 2 + s2
          x2 = xs_ref[i]
          ccol = jax.lax.broadcasted_iota(jnp.int32, (1, _HW), 1) % _W
          zero = jnp.bfloat16(0)
          x2l = jnp.where(ccol == _W - 1, zero, x2)
          x2r = jnp.where(ccol == 0, zero, x2)
          xe = jnp.pad(x2, ((0, 0), (_PAD, _PAD)))
          xel = jnp.pad(x2l, ((0, 0), (_PAD, _PAD)))
          xer = jnp.pad(x2r, ((0, 0), (_PAD, _PAD)))
          srcs = (xel, xe, xer)

          acc_a = jnp.zeros((_C, _HW), jnp.float32)
          acc_b = jnp.zeros((_C, _HW), jnp.float32)
          for dy in range(3):
              for dx in range(3):
                  s = (dy - 1) * _W + (dx - 1)
                  xsl = srcs[dx][:, _PAD + s:_PAD + s + _HW]
                  acc_a = acc_a + jnp.dot(wt_ref[0, dy * 3 + dx], xsl,
                                          preferred_element_type=jnp.float32)
                  acc_b = acc_b + jnp.dot(wt_ref[1, dy * 3 + dx], xsl,
                                          preferred_element_type=jnp.float32)
          out_ref[s2, :_C] = acc_a + bsel_ref[0]
          out_ref[s2, _C:] = acc_b + bsel_ref[1]


def kernel(x, W_router, W_conv, b_conv):
    x3 = x.reshape(_B, _C, _HW)
    wr3 = W_router.reshape(_E, _C, _HW)
    wc2 = W_conv.reshape(_EC, _WC)

    loss2, sel2, out_raw = pl.pallas_call(
        _fused_kernel,
        grid=(_B // 2 + _NC,),
        in_specs=[
            pl.BlockSpec((_B, _CC, _HW),
                         lambda b: (0, jnp.minimum(b, _NC - 1), 0)),
            pl.BlockSpec((_E, _CC, _HW),
                         lambda b: (0, jnp.minimum(b, _NC - 1), 0)),
            pl.BlockSpec((_EC, _WC), lambda b: (0, 0)),
            pl.BlockSpec((_E, _C), lambda b: (0, 0)),
        ],
        out_specs=[
            pl.BlockSpec((1, 1), lambda b: (0, 0)),
            pl.BlockSpec((1, 2), lambda b: (0, 0)),
            pl.BlockSpec((2, 2 * _C, _HW),
                         lambda b: (jnp.maximum(b - _NC, 0), 0, 0)),
        ],
        out_shape=[
            jax.ShapeDtypeStruct((1, 1), jnp.float32),
            jax.ShapeDtypeStruct((1, 2), jnp.int32),
            jax.ShapeDtypeStruct((_B, 2 * _C, _HW), jnp.float32),
        ],
        scratch_shapes=[
            pltpu.VMEM((_B, _C, _HW), jnp.bfloat16),
            pltpu.VMEM((2, 9, _C, _C), jnp.bfloat16),
            pltpu.VMEM((2, _C, 1), jnp.float32),
            pltpu.VMEM((_B, _E), jnp.float32),
        ],
    )(x3, wr3, wc2, b_conv)

    router_loss = loss2.reshape(())
    expert_outputs = out_raw.reshape(_B, 2 * _C, _H, _W)
    return expert_outputs, router_loss
